# baseline (speedup 1.0000x reference)
import functools

import jax
import jax.numpy as jnp
from jax.experimental import pallas as pl
from jax.experimental.pallas import tpu as pltpu

_H = 112
_CP = 4
_NEG = -1e30
_PREC = jax.lax.Precision.DEFAULT
_F = jnp.float32


def _conv1_kernel(x_ref, w_ref, b_ref, o_ref):
    def body(ob, _):
        o0 = ob * 4
        accs = [jnp.zeros((224, 224), _F) + b_ref[o0 + t] for t in range(4)]
        for p in range(3):
            for k in range(9):
                dy, dx = divmod(k, 3)
                sl = x_ref[p, dy:dy + 224, dx:dx + 224]
                for t in range(4):
                    accs[t] += sl * w_ref[o0 + t, p * 9 + k]
        for t in range(4):
            o_ref[o0 + t] = jnp.maximum(accs[t], 0.0)
        return 0

    jax.lax.fori_loop(0, 16, body, 0, unroll=False)


def _conv_acc(x_ref, w_ref, b_ref, i, *, rows, out_w, d, in_rp, mmdt):
    w2 = x_ref.shape[1]
    cin = x_ref.shape[-1]
    cout = w_ref.shape[-1]
    slabs = [x_ref[pl.ds(in_rp - d + i * rows + dy * d, rows), :, :]
             .astype(mmdt).reshape(rows * w2, cin) for dy in range(3)]
    acc = jnp.zeros((rows, out_w, cout), _F) + b_ref[...][None]
    for dx in range(3):
        y = sum(jnp.dot(slabs[dy], w_ref[dy * 3 + dx],
                        preferred_element_type=_F,
                        precision=_PREC) for dy in range(3))
        c0 = _CP - d + dx * d
        acc += y.reshape(rows, w2, cout)[:, c0:c0 + out_w, :]
    return acc


def _store_padded(o_ref, val, out_w):
    cout = o_ref.shape[-1]
    rows = o_ref.shape[0]
    dt = o_ref.dtype
    o_ref[:, _CP:_CP + out_w, :] = val.astype(dt)
    o_ref[:, 0:_CP, :] = jnp.zeros((rows, _CP, cout), dt)
    o_ref[:, _CP + out_w:, :] = jnp.zeros(
        (rows, o_ref.shape[1] - _CP - out_w, cout), dt)


def _dconv_kernel(x_ref, w_ref, b_ref, o_ref, *, rows, out_w, d, in_rp, G):
    j = pl.program_id(0)

    @pl.when((j == 0) | (j == G + 1))
    def _():
        o_ref[...] = jnp.zeros(o_ref.shape, o_ref.dtype)

    @pl.when((j > 0) & (j < G + 1))
    def _():
        acc = _conv_acc(x_ref, w_ref, b_ref, j - 1, rows=rows, out_w=out_w,
                        d=d, in_rp=in_rp, mmdt=_F)
        _store_padded(o_ref, jnp.maximum(acc, 0.0), out_w)


def _dconv_sent_kernel(x_ref, w_ref, b_ref, m_ref, o_ref, *, rows, d, in_rp,
                       G):
    j = pl.program_id(0)

    @pl.when((j == 0) | (j == G + 1))
    def _():
        o_ref[...] = jnp.zeros(o_ref.shape, o_ref.dtype)

    @pl.when((j > 0) & (j < G + 1))
    def _():
        acc = _conv_acc(x_ref, w_ref, b_ref, j - 1, rows=rows, out_w=_H,
                        d=d, in_rp=in_rp, mmdt=_F)
        val = jnp.where(m_ref[...] > 0, jnp.maximum(acc, 0.0), _NEG)
        _store_padded(o_ref, val, _H)


def _gconv_kernel(x_ref, w_ref, b_ref, m_ref, o_ref, *, rows, d, in_rp, G,
                  mmdt, sentinel):
    j = pl.program_id(0)

    @pl.when((j == 0) | (j == G + 1))
    def _():
        o_ref[...] = jnp.zeros(o_ref.shape, o_ref.dtype)

    @pl.when((j > 0) & (j < G + 1))
    def _():
        acc = _conv_acc(x_ref, w_ref, b_ref, j - 1, rows=rows, out_w=_H,
                        d=d, in_rp=in_rp, mmdt=mmdt)
        if sentinel:
            val = jnp.where(m_ref[...] > 0, jnp.maximum(acc, 0.0), _NEG)
        else:
            val = jnp.maximum(acc, 0.0) * m_ref[...]
        _store_padded(o_ref, val, _H)


def _gc89_kernel(x_ref, w8_ref, b8_ref, w9_ref, b9_ref, mf_ref, m_ref, o_ref,
                 s_ref, *, rows, G):
    j = pl.program_id(0)
    bf = jnp.bfloat16

    @pl.when(j == 0)
    def _():
        zrow = jnp.zeros((rows, s_ref.shape[1], s_ref.shape[2]), bf)
        s_ref[0:rows] = zrow
        s_ref[s_ref.shape[0] - rows:] = zrow

        def gb(t, _):
            acc = _conv_acc(x_ref, w8_ref, b8_ref, t, rows=rows, out_w=_H,
                            d=4, in_rp=16, mmdt=_F)
            m = mf_ref[pl.ds(t * rows, rows), :, :].astype(_F)
            val = (jnp.maximum(acc, 0.0) * m).astype(bf)
            r0 = rows + t * rows
            s_ref[pl.ds(r0, rows), _CP:_CP + _H, :] = val
            s_ref[pl.ds(r0, rows), 0:_CP, :] = jnp.zeros((rows, _CP, 512), bf)
            s_ref[pl.ds(r0, rows), _CP + _H:, :] = jnp.zeros(
                (rows, _CP, 512), bf)
            return 0

        jax.lax.fori_loop(0, G, gb, 0, unroll=False)

    @pl.when((j == 0) | (j == G + 1))
    def _():
        o_ref[...] = jnp.zeros(o_ref.shape, o_ref.dtype)

    @pl.when((j > 0) & (j < G + 1))
    def _():
        acc = _conv_acc(s_ref, w9_ref, b9_ref, j - 1, rows=rows, out_w=_H,
                        d=4, in_rp=rows, mmdt=jnp.bfloat16)
        _store_padded(o_ref, jnp.maximum(acc, 0.0) * m_ref[...], _H)


def _gconv_last_kernel(x_ref, w_ref, b_ref, m_ref, o_ref, *, rows, d, in_rp,
                       mmdt):
    i = pl.program_id(0)
    acc = _conv_acc(x_ref, w_ref, b_ref, i, rows=rows, out_w=_H, d=d,
                    in_rp=in_rp, mmdt=mmdt)
    o_ref[...] = jnp.maximum(acc, 0.0) * m_ref[...]


def _maxpool_kernel(x_ref, o_ref, *, rows, in_rp, G):
    j = pl.program_id(0)

    @pl.when((j == 0) | (j == G + 1))
    def _():
        o_ref[...] = jnp.zeros(o_ref.shape, o_ref.dtype)

    @pl.when((j > 0) & (j < G + 1))
    def _():
        t0 = (j - 1) * rows
        a00 = x_ref[pl.Slice(in_rp + 2 * t0, rows, 2),
                    pl.Slice(_CP, _H, 2), :]
        a01 = x_ref[pl.Slice(in_rp + 2 * t0, rows, 2),
                    pl.Slice(_CP + 1, _H, 2), :]
        a10 = x_ref[pl.Slice(in_rp + 2 * t0 + 1, rows, 2),
                    pl.Slice(_CP, _H, 2), :]
        a11 = x_ref[pl.Slice(in_rp + 2 * t0 + 1, rows, 2),
                    pl.Slice(_CP + 1, _H, 2), :]
        _store_padded(
            o_ref, jnp.maximum(jnp.maximum(a00, a01), jnp.maximum(a10, a11)),
            _H)


def _pool1_kernel(x_ref, m_ref, o_ref, *, rows, d, in_rp, G):
    j = pl.program_id(0)

    @pl.when((j == 0) | (j == G + 1))
    def _():
        o_ref[...] = jnp.zeros(o_ref.shape, o_ref.dtype)

    @pl.when((j > 0) & (j < G + 1))
    def _():
        t0 = (j - 1) * rows
        best = jnp.full((rows, _H, x_ref.shape[-1]), _NEG, _F)
        for dy in (0, d):
            for dx in (0, d):
                best = jnp.maximum(
                    best, x_ref[pl.ds(in_rp + t0 + dy, rows),
                                _CP + dx:_CP + dx + _H, :])
        best = jnp.where(best < -1e29, 0.0, best)
        _store_padded(o_ref, best * m_ref[...], _H)


def _pool2_kernel(x_ref, m_ref, o_ref, *, rows, d, G):
    j = pl.program_id(0)

    @pl.when((j == 0) | (j == G + 1))
    def _():
        o_ref[...] = jnp.zeros(o_ref.shape, o_ref.dtype)

    @pl.when((j > 0) & (j < G + 1))
    def _():
        t0 = (j - 1) * rows
        best = jnp.full((rows, _H, x_ref.shape[-1]), _NEG, _F)
        for dy in (0, d):
            for dx in (0, d):
                best = jnp.maximum(
                    best,
                    x_ref[pl.ds(t0 + dy, rows), dx:dx + _H, :].astype(_F))
        best = jnp.where(best < -1e29, 0.0, best)
        _store_padded(o_ref, best * m_ref[...], _H)


def _full(shape):
    n = len(shape)
    return pl.BlockSpec(shape, lambda j: (0,) * n)


def _taps(w):
    return jnp.transpose(w, (2, 3, 1, 0)).reshape(9, w.shape[1], w.shape[0])


def _padded_call(kern, ins, specs, *, rows, out_h, out_w, cout, out_dt=_F):
    G = out_h // rows
    return pl.pallas_call(
        kern,
        grid=(G + 2,),
        in_specs=specs,
        out_specs=pl.BlockSpec((rows, out_w + 2 * _CP, cout),
                               lambda j: (j, 0, 0)),
        out_shape=jax.ShapeDtypeStruct(
            (out_h + 2 * rows, out_w + 2 * _CP, cout), out_dt),
    )(*ins)


def _mrow_spec(rows):
    return pl.BlockSpec((rows, _H, 1), lambda j: (j, 0, 0))


def kernel(batch, pooling_mask, w1, b1, w2, b2, w3, b3, w4, b4, w5, b5,
           w6, b6, w7, b7, w8, b8, w9, b9, w10, b10):
    mask3 = pooling_mask[0].astype(_F)[:, :, None]
    mp16 = jnp.pad(mask3, ((16, 16), (0, 0), (0, 0)))

    x0p = jnp.pad(batch[0], ((0, 0), (1, 1), (1, 1)))
    x1pl = pl.pallas_call(
        _conv1_kernel,
        in_specs=[pl.BlockSpec(memory_space=pltpu.VMEM),
                  pl.BlockSpec(memory_space=pltpu.SMEM),
                  pl.BlockSpec(memory_space=pltpu.SMEM)],
        out_specs=pl.BlockSpec(memory_space=pltpu.VMEM),
        out_shape=jax.ShapeDtypeStruct((64, 224, 224), _F),
    )(x0p, w1.reshape(64, 27), b1)
    x1p = jnp.pad(jnp.transpose(x1pl, (1, 2, 0)),
                  ((1, 1), (_CP, _CP), (0, 0)))

    x2f = _padded_call(
        functools.partial(_dconv_kernel, rows=16, out_w=224, d=1, in_rp=1,
                          G=14),
        (x1p, _taps(w2), b2.reshape(1, -1)),
        [_full(x1p.shape), _full((9, 64, 64)), _full((1, 64))],
        rows=16, out_h=224, out_w=224, cout=64)

    x2 = _padded_call(
        functools.partial(_maxpool_kernel, rows=16, in_rp=16, G=7),
        (x2f,), [_full(x2f.shape)],
        rows=16, out_h=_H, out_w=_H, cout=64,
        out_dt=jnp.bfloat16)

    x3 = _padded_call(
        functools.partial(_dconv_kernel, rows=16, out_w=_H, d=1, in_rp=16,
                          G=7),
        (x2, _taps(w3), b3.reshape(1, -1)),
        [_full(x2.shape), _full((9, 64, 128)), _full((1, 128))],
        rows=16, out_h=_H, out_w=_H, cout=128,
        out_dt=jnp.bfloat16)

    x4 = _padded_call(
        functools.partial(_dconv_sent_kernel, rows=16, d=1, in_rp=16, G=7),
        (x3, _taps(w4), b4.reshape(1, -1), mp16),
        [_full(x3.shape), _full((9, 128, 128)), _full((1, 128)),
         _mrow_spec(16)],
        rows=16, out_h=_H, out_w=_H, cout=128,
        out_dt=jnp.bfloat16)

    n1 = _padded_call(
        functools.partial(_pool1_kernel, rows=16, d=1, in_rp=16, G=7),
        (x4, mp16), [_full(x4.shape), _mrow_spec(16)],
        rows=16, out_h=_H, out_w=_H, cout=128,
        out_dt=jnp.bfloat16)

    def gconv_pad(x, w, b, mp, rows, d, in_rp, cin, cout, mmdt=_F,
                  sentinel=False, out_dt=_F):
        return _padded_call(
            functools.partial(_gconv_kernel, rows=rows, d=d, in_rp=in_rp,
                              G=_H // rows, mmdt=mmdt, sentinel=sentinel),
            (x, w.astype(mmdt), b.reshape(1, -1), mp),
            [_full(x.shape), _full((9, cin, cout)), _full((1, cout)),
             _mrow_spec(rows)],
            rows=rows, out_h=_H, out_w=_H, cout=cout, out_dt=out_dt)

    bf = jnp.bfloat16
    g5 = gconv_pad(n1, w5, b5, mp16, 16, 2, 16, 128, 256, out_dt=bf)
    g6 = gconv_pad(g5, w6, b6, mp16, 16, 2, 16, 256, 256, out_dt=bf)
    g7 = gconv_pad(g6, w7, b7, mp16, 16, 2, 16, 256, 256, sentinel=True,
                   out_dt=bf)

    g7i = g7[16:16 + _H, _CP:_CP + _H, :]
    g7e = jnp.pad(g7i, ((0, 8), (0, 8), (0, 0)), mode="edge")
    n2 = _padded_call(
        functools.partial(_pool2_kernel, rows=16, d=2, G=7),
        (g7e, mp16), [_full(g7e.shape), _mrow_spec(16)],
        rows=16, out_h=_H, out_w=_H, cout=256,
        out_dt=jnp.bfloat16)

    mp8 = jnp.pad(mask3, ((8, 8), (0, 0), (0, 0)))
    g9 = pl.pallas_call(
        functools.partial(_gc89_kernel, rows=8, G=14),
        grid=(16,),
        in_specs=[_full(n2.shape), _full((9, 256, 512)), _full((1, 512)),
                  _full((9, 512, 512)), _full((1, 512)),
                  _full((_H, _H, 1)), _mrow_spec(8)],
        out_specs=pl.BlockSpec((8, _H + 2 * _CP, 512), lambda j: (j, 0, 0)),
        out_shape=jax.ShapeDtypeStruct((_H + 16, _H + 2 * _CP, 512), bf),
        scratch_shapes=[pltpu.VMEM((_H + 16, _H + 2 * _CP, 512), bf)],
    )(n2, w8, b8.reshape(1, -1), w9.astype(bf), b9.reshape(1, -1),
      mask3.astype(bf), mp8)

    g10 = pl.pallas_call(
        functools.partial(_gconv_last_kernel, rows=16, d=4, in_rp=8,
                          mmdt=jnp.bfloat16),
        grid=(7,),
        in_specs=[_full(g9.shape), _full((9, 512, 512)), _full((1, 512)),
                  pl.BlockSpec((16, _H, 1), lambda i: (i, 0, 0))],
        out_specs=pl.BlockSpec((16, _H, 512), lambda i: (i, 0, 0)),
        out_shape=jax.ShapeDtypeStruct((_H, _H, 512), _F),
    )(g9, w10.astype(jnp.bfloat16), b10.reshape(1, -1), mask3)

    return jnp.transpose(g10, (2, 0, 1))[None]


# speedup vs baseline: 1.0096x; 1.0096x over previous
import functools

import jax
import jax.numpy as jnp
from jax.experimental import pallas as pl
from jax.experimental.pallas import tpu as pltpu

_H = 112
_CP = 4
_NEG = -1e30
_PREC = jax.lax.Precision.DEFAULT
_F = jnp.float32


def _conv1_kernel(x_ref, w_ref, b_ref, o_ref):
    def body(ob, _):
        o0 = ob * 4
        accs = [jnp.zeros((224, 224), _F) + b_ref[o0 + t] for t in range(4)]
        for p in range(3):
            for k in range(9):
                dy, dx = divmod(k, 3)
                sl = x_ref[p, dy:dy + 224, dx:dx + 224]
                for t in range(4):
                    accs[t] += sl * w_ref[o0 + t, p * 9 + k]
        for t in range(4):
            o_ref[o0 + t] = jnp.maximum(accs[t], 0.0)
        return 0

    jax.lax.fori_loop(0, 16, body, 0, unroll=False)


def _conv_acc(x_ref, w_ref, b_ref, i, *, rows, out_w, d, in_rp, mmdt):
    w2 = x_ref.shape[1]
    cin = x_ref.shape[-1]
    cout = w_ref.shape[-1]
    slabs = [x_ref[pl.ds(in_rp - d + i * rows + dy * d, rows), :, :]
             .astype(mmdt).reshape(rows * w2, cin) for dy in range(3)]
    acc = jnp.zeros((rows, out_w, cout), _F) + b_ref[...][None]
    for dx in range(3):
        y = sum(jnp.dot(slabs[dy], w_ref[dy * 3 + dx],
                        preferred_element_type=_F,
                        precision=_PREC) for dy in range(3))
        c0 = _CP - d + dx * d
        acc += y.reshape(rows, w2, cout)[:, c0:c0 + out_w, :]
    return acc


def _store_padded(o_ref, val, out_w):
    cout = o_ref.shape[-1]
    rows = o_ref.shape[0]
    dt = o_ref.dtype
    o_ref[:, _CP:_CP + out_w, :] = val.astype(dt)
    o_ref[:, 0:_CP, :] = jnp.zeros((rows, _CP, cout), dt)
    o_ref[:, _CP + out_w:, :] = jnp.zeros(
        (rows, o_ref.shape[1] - _CP - out_w, cout), dt)


def _dconv_kernel(x_ref, w_ref, b_ref, o_ref, *, rows, out_w, d, in_rp, G):
    j = pl.program_id(0)

    @pl.when((j == 0) | (j == G + 1))
    def _():
        o_ref[...] = jnp.zeros(o_ref.shape, o_ref.dtype)

    @pl.when((j > 0) & (j < G + 1))
    def _():
        acc = _conv_acc(x_ref, w_ref, b_ref, j - 1, rows=rows, out_w=out_w,
                        d=d, in_rp=in_rp, mmdt=_F)
        _store_padded(o_ref, jnp.maximum(acc, 0.0), out_w)


def _dconv_sent_kernel(x_ref, w_ref, b_ref, m_ref, o_ref, *, rows, d, in_rp,
                       G):
    j = pl.program_id(0)

    @pl.when((j == 0) | (j == G + 1))
    def _():
        o_ref[...] = jnp.zeros(o_ref.shape, o_ref.dtype)

    @pl.when((j > 0) & (j < G + 1))
    def _():
        acc = _conv_acc(x_ref, w_ref, b_ref, j - 1, rows=rows, out_w=_H,
                        d=d, in_rp=in_rp, mmdt=_F)
        val = jnp.where(m_ref[...] > 0, jnp.maximum(acc, 0.0), _NEG)
        _store_padded(o_ref, val, _H)


def _gconv_kernel(x_ref, w_ref, b_ref, m_ref, o_ref, *, rows, d, in_rp, G,
                  mmdt, sentinel):
    j = pl.program_id(0)

    @pl.when((j == 0) | (j == G + 1))
    def _():
        o_ref[...] = jnp.zeros(o_ref.shape, o_ref.dtype)

    @pl.when((j > 0) & (j < G + 1))
    def _():
        acc = _conv_acc(x_ref, w_ref, b_ref, j - 1, rows=rows, out_w=_H,
                        d=d, in_rp=in_rp, mmdt=mmdt)
        if sentinel:
            val = jnp.where(m_ref[...] > 0, jnp.maximum(acc, 0.0), _NEG)
        else:
            val = jnp.maximum(acc, 0.0) * m_ref[...]
        _store_padded(o_ref, val, _H)


def _gconv_last_kernel(x_ref, w_ref, b_ref, m_ref, o_ref, *, rows, d, in_rp,
                       mmdt):
    i = pl.program_id(0)
    acc = _conv_acc(x_ref, w_ref, b_ref, i, rows=rows, out_w=_H, d=d,
                    in_rp=in_rp, mmdt=mmdt)
    o_ref[...] = jnp.maximum(acc, 0.0) * m_ref[...]


def _maxpool_kernel(x_ref, o_ref, *, rows, in_rp, G):
    j = pl.program_id(0)

    @pl.when((j == 0) | (j == G + 1))
    def _():
        o_ref[...] = jnp.zeros(o_ref.shape, o_ref.dtype)

    @pl.when((j > 0) & (j < G + 1))
    def _():
        t0 = (j - 1) * rows
        a00 = x_ref[pl.Slice(in_rp + 2 * t0, rows, 2),
                    pl.Slice(_CP, _H, 2), :]
        a01 = x_ref[pl.Slice(in_rp + 2 * t0, rows, 2),
                    pl.Slice(_CP + 1, _H, 2), :]
        a10 = x_ref[pl.Slice(in_rp + 2 * t0 + 1, rows, 2),
                    pl.Slice(_CP, _H, 2), :]
        a11 = x_ref[pl.Slice(in_rp + 2 * t0 + 1, rows, 2),
                    pl.Slice(_CP + 1, _H, 2), :]
        _store_padded(
            o_ref, jnp.maximum(jnp.maximum(a00, a01), jnp.maximum(a10, a11)),
            _H)


def _pool1_kernel(x_ref, m_ref, o_ref, *, rows, d, in_rp, G):
    j = pl.program_id(0)

    @pl.when((j == 0) | (j == G + 1))
    def _():
        o_ref[...] = jnp.zeros(o_ref.shape, o_ref.dtype)

    @pl.when((j > 0) & (j < G + 1))
    def _():
        t0 = (j - 1) * rows
        best = jnp.full((rows, _H, x_ref.shape[-1]), _NEG, _F)
        for dy in (0, d):
            for dx in (0, d):
                best = jnp.maximum(
                    best, x_ref[pl.ds(in_rp + t0 + dy, rows),
                                _CP + dx:_CP + dx + _H, :])
        best = jnp.where(best < -1e29, 0.0, best)
        _store_padded(o_ref, best * m_ref[...], _H)


def _pool2_kernel(x_ref, m_ref, o_ref, *, rows, d, G):
    j = pl.program_id(0)

    @pl.when((j == 0) | (j == G + 1))
    def _():
        o_ref[...] = jnp.zeros(o_ref.shape, o_ref.dtype)

    @pl.when((j > 0) & (j < G + 1))
    def _():
        t0 = (j - 1) * rows
        best = jnp.full((rows, _H, x_ref.shape[-1]), _NEG, _F)
        for dy in (0, d):
            for dx in (0, d):
                best = jnp.maximum(
                    best,
                    x_ref[pl.ds(t0 + dy, rows), dx:dx + _H, :].astype(_F))
        best = jnp.where(best < -1e29, 0.0, best)
        _store_padded(o_ref, best * m_ref[...], _H)


def _full(shape):
    n = len(shape)
    return pl.BlockSpec(shape, lambda j: (0,) * n)


def _taps(w):
    return jnp.transpose(w, (2, 3, 1, 0)).reshape(9, w.shape[1], w.shape[0])


def _padded_call(kern, ins, specs, *, rows, out_h, out_w, cout, out_dt=_F):
    G = out_h // rows
    return pl.pallas_call(
        kern,
        grid=(G + 2,),
        in_specs=specs,
        out_specs=pl.BlockSpec((rows, out_w + 2 * _CP, cout),
                               lambda j: (j, 0, 0)),
        out_shape=jax.ShapeDtypeStruct(
            (out_h + 2 * rows, out_w + 2 * _CP, cout), out_dt),
    )(*ins)


def _mrow_spec(rows):
    return pl.BlockSpec((rows, _H, 1), lambda j: (j, 0, 0))


def kernel(batch, pooling_mask, w1, b1, w2, b2, w3, b3, w4, b4, w5, b5,
           w6, b6, w7, b7, w8, b8, w9, b9, w10, b10):
    mask3 = pooling_mask[0].astype(_F)[:, :, None]
    mp16 = jnp.pad(mask3, ((16, 16), (0, 0), (0, 0)))

    x0p = jnp.pad(batch[0], ((0, 0), (1, 1), (1, 1)))
    x1pl = pl.pallas_call(
        _conv1_kernel,
        in_specs=[pl.BlockSpec(memory_space=pltpu.VMEM),
                  pl.BlockSpec(memory_space=pltpu.SMEM),
                  pl.BlockSpec(memory_space=pltpu.SMEM)],
        out_specs=pl.BlockSpec(memory_space=pltpu.VMEM),
        out_shape=jax.ShapeDtypeStruct((64, 224, 224), _F),
    )(x0p, w1.reshape(64, 27), b1)
    x1p = jnp.pad(jnp.transpose(x1pl, (1, 2, 0)),
                  ((1, 1), (_CP, _CP), (0, 0)))

    x2f = _padded_call(
        functools.partial(_dconv_kernel, rows=16, out_w=224, d=1, in_rp=1,
                          G=14),
        (x1p, _taps(w2), b2.reshape(1, -1)),
        [_full(x1p.shape), _full((9, 64, 64)), _full((1, 64))],
        rows=16, out_h=224, out_w=224, cout=64)

    x2 = _padded_call(
        functools.partial(_maxpool_kernel, rows=16, in_rp=16, G=7),
        (x2f,), [_full(x2f.shape)],
        rows=16, out_h=_H, out_w=_H, cout=64,
        out_dt=jnp.bfloat16)

    x3 = _padded_call(
        functools.partial(_dconv_kernel, rows=16, out_w=_H, d=1, in_rp=16,
                          G=7),
        (x2, _taps(w3), b3.reshape(1, -1)),
        [_full(x2.shape), _full((9, 64, 128)), _full((1, 128))],
        rows=16, out_h=_H, out_w=_H, cout=128,
        out_dt=jnp.bfloat16)

    x4 = _padded_call(
        functools.partial(_dconv_sent_kernel, rows=16, d=1, in_rp=16, G=7),
        (x3, _taps(w4), b4.reshape(1, -1), mp16),
        [_full(x3.shape), _full((9, 128, 128)), _full((1, 128)),
         _mrow_spec(16)],
        rows=16, out_h=_H, out_w=_H, cout=128,
        out_dt=jnp.bfloat16)

    n1 = _padded_call(
        functools.partial(_pool1_kernel, rows=16, d=1, in_rp=16, G=7),
        (x4, mp16), [_full(x4.shape), _mrow_spec(16)],
        rows=16, out_h=_H, out_w=_H, cout=128,
        out_dt=jnp.bfloat16)

    def gconv_pad(x, w, b, mp, rows, d, in_rp, cin, cout, mmdt=_F,
                  sentinel=False, out_dt=_F):
        return _padded_call(
            functools.partial(_gconv_kernel, rows=rows, d=d, in_rp=in_rp,
                              G=_H // rows, mmdt=mmdt, sentinel=sentinel),
            (x, w.astype(mmdt), b.reshape(1, -1), mp),
            [_full(x.shape), _full((9, cin, cout)), _full((1, cout)),
             _mrow_spec(rows)],
            rows=rows, out_h=_H, out_w=_H, cout=cout, out_dt=out_dt)

    bf = jnp.bfloat16
    g5 = gconv_pad(n1, w5, b5, mp16, 16, 2, 16, 128, 256, out_dt=bf)
    g6 = gconv_pad(g5, w6, b6, mp16, 16, 2, 16, 256, 256, out_dt=bf)
    g7 = gconv_pad(g6, w7, b7, mp16, 16, 2, 16, 256, 256, sentinel=True,
                   out_dt=bf)

    g7i = g7[16:16 + _H, _CP:_CP + _H, :]
    g7e = jnp.pad(g7i, ((0, 8), (0, 8), (0, 0)), mode="edge")
    n2 = _padded_call(
        functools.partial(_pool2_kernel, rows=16, d=2, G=7),
        (g7e, mp16), [_full(g7e.shape), _mrow_spec(16)],
        rows=16, out_h=_H, out_w=_H, cout=256,
        out_dt=jnp.bfloat16)

    g8 = gconv_pad(n2, w8, b8, mp16, 16, 4, 16, 256, 512, out_dt=bf)
    g9 = gconv_pad(g8, w9, b9, mp16, 16, 4, 16, 512, 512,
                   mmdt=jnp.bfloat16, out_dt=jnp.bfloat16)

    g10 = pl.pallas_call(
        functools.partial(_gconv_last_kernel, rows=16, d=4, in_rp=16,
                          mmdt=jnp.bfloat16),
        grid=(7,),
        in_specs=[_full(g9.shape), _full((9, 512, 512)), _full((1, 512)),
                  pl.BlockSpec((16, _H, 1), lambda i: (i, 0, 0))],
        out_specs=pl.BlockSpec((16, _H, 512), lambda i: (i, 0, 0)),
        out_shape=jax.ShapeDtypeStruct((_H, _H, 512), _F),
    )(g9, w10.astype(jnp.bfloat16), b10.reshape(1, -1), mask3)

    return jnp.transpose(g10, (2, 0, 1))[None]


# speedup vs baseline: 1.0378x; 1.0280x over previous
import functools

import jax
import jax.numpy as jnp
from jax.experimental import pallas as pl
from jax.experimental.pallas import tpu as pltpu

_H = 112
_CP = 4
_NEG = -1e30
_PREC = jax.lax.Precision.DEFAULT
_F = jnp.float32


def _conv1_kernel(x_ref, w_ref, b_ref, o_ref):
    def body(ob, _):
        o0 = ob * 4
        accs = [jnp.zeros((224, 224), _F) + b_ref[o0 + t] for t in range(4)]
        for p in range(3):
            for k in range(9):
                dy, dx = divmod(k, 3)
                sl = x_ref[p, dy:dy + 224, dx:dx + 224]
                for t in range(4):
                    accs[t] += sl * w_ref[o0 + t, p * 9 + k]
        for t in range(4):
            o_ref[o0 + t] = jnp.maximum(accs[t], 0.0).astype(o_ref.dtype)
        return 0

    jax.lax.fori_loop(0, 16, body, 0, unroll=False)


def _conv_acc(x_ref, w_ref, b_ref, i, *, rows, out_w, d, in_rp, mmdt):
    w2 = x_ref.shape[1]
    cin = x_ref.shape[-1]
    cout = w_ref.shape[-1]
    slabs = [x_ref[pl.ds(in_rp - d + i * rows + dy * d, rows), :, :]
             .astype(mmdt).reshape(rows * w2, cin) for dy in range(3)]
    acc = jnp.zeros((rows, out_w, cout), _F) + b_ref[...][None]
    for dx in range(3):
        y = sum(jnp.dot(slabs[dy], w_ref[dy * 3 + dx],
                        preferred_element_type=_F,
                        precision=_PREC) for dy in range(3))
        c0 = _CP - d + dx * d
        acc += y.reshape(rows, w2, cout)[:, c0:c0 + out_w, :]
    return acc


def _store_padded(o_ref, val, out_w):
    cout = o_ref.shape[-1]
    rows = o_ref.shape[0]
    dt = o_ref.dtype
    o_ref[:, _CP:_CP + out_w, :] = val.astype(dt)
    o_ref[:, 0:_CP, :] = jnp.zeros((rows, _CP, cout), dt)
    o_ref[:, _CP + out_w:, :] = jnp.zeros(
        (rows, o_ref.shape[1] - _CP - out_w, cout), dt)


def _dconv_kernel(x_ref, w_ref, b_ref, o_ref, *, rows, out_w, d, in_rp, G):
    j = pl.program_id(0)

    @pl.when((j == 0) | (j == G + 1))
    def _():
        o_ref[...] = jnp.zeros(o_ref.shape, o_ref.dtype)

    @pl.when((j > 0) & (j < G + 1))
    def _():
        acc = _conv_acc(x_ref, w_ref, b_ref, j - 1, rows=rows, out_w=out_w,
                        d=d, in_rp=in_rp, mmdt=_F)
        _store_padded(o_ref, jnp.maximum(acc, 0.0), out_w)


def _dconv_sent_kernel(x_ref, w_ref, b_ref, m_ref, o_ref, *, rows, d, in_rp,
                       G):
    j = pl.program_id(0)

    @pl.when((j == 0) | (j == G + 1))
    def _():
        o_ref[...] = jnp.zeros(o_ref.shape, o_ref.dtype)

    @pl.when((j > 0) & (j < G + 1))
    def _():
        acc = _conv_acc(x_ref, w_ref, b_ref, j - 1, rows=rows, out_w=_H,
                        d=d, in_rp=in_rp, mmdt=_F)
        val = jnp.where(m_ref[...] > 0, jnp.maximum(acc, 0.0), _NEG)
        _store_padded(o_ref, val, _H)


def _gconv_kernel(x_ref, w_ref, b_ref, m_ref, o_ref, *, rows, d, in_rp, G,
                  mmdt, sentinel):
    j = pl.program_id(0)

    @pl.when((j == 0) | (j == G + 1))
    def _():
        o_ref[...] = jnp.zeros(o_ref.shape, o_ref.dtype)

    @pl.when((j > 0) & (j < G + 1))
    def _():
        acc = _conv_acc(x_ref, w_ref, b_ref, j - 1, rows=rows, out_w=_H,
                        d=d, in_rp=in_rp, mmdt=mmdt)
        if sentinel:
            val = jnp.where(m_ref[...] > 0, jnp.maximum(acc, 0.0), _NEG)
        else:
            val = jnp.maximum(acc, 0.0) * m_ref[...]
        _store_padded(o_ref, val, _H)


def _gconv_last_kernel(x_ref, w_ref, b_ref, m_ref, o_ref, *, rows, d, in_rp,
                       mmdt):
    i = pl.program_id(0)
    acc = _conv_acc(x_ref, w_ref, b_ref, i, rows=rows, out_w=_H, d=d,
                    in_rp=in_rp, mmdt=mmdt)
    o_ref[...] = jnp.maximum(acc, 0.0) * m_ref[...]


def _maxpool_kernel(x_ref, o_ref, *, rows, in_rp, G):
    j = pl.program_id(0)

    @pl.when((j == 0) | (j == G + 1))
    def _():
        o_ref[...] = jnp.zeros(o_ref.shape, o_ref.dtype)

    @pl.when((j > 0) & (j < G + 1))
    def _():
        t0 = (j - 1) * rows
        a00 = x_ref[pl.Slice(in_rp + 2 * t0, rows, 2),
                    pl.Slice(_CP, _H, 2), :]
        a01 = x_ref[pl.Slice(in_rp + 2 * t0, rows, 2),
                    pl.Slice(_CP + 1, _H, 2), :]
        a10 = x_ref[pl.Slice(in_rp + 2 * t0 + 1, rows, 2),
                    pl.Slice(_CP, _H, 2), :]
        a11 = x_ref[pl.Slice(in_rp + 2 * t0 + 1, rows, 2),
                    pl.Slice(_CP + 1, _H, 2), :]
        _store_padded(
            o_ref, jnp.maximum(jnp.maximum(a00, a01), jnp.maximum(a10, a11)),
            _H)


def _pool1_kernel(x_ref, m_ref, o_ref, *, rows, d, in_rp, G):
    j = pl.program_id(0)

    @pl.when((j == 0) | (j == G + 1))
    def _():
        o_ref[...] = jnp.zeros(o_ref.shape, o_ref.dtype)

    @pl.when((j > 0) & (j < G + 1))
    def _():
        t0 = (j - 1) * rows
        best = jnp.full((rows, _H, x_ref.shape[-1]), _NEG, _F)
        for dy in (0, d):
            for dx in (0, d):
                best = jnp.maximum(
                    best, x_ref[pl.ds(in_rp + t0 + dy, rows),
                                _CP + dx:_CP + dx + _H, :])
        best = jnp.where(best < -1e29, 0.0, best)
        _store_padded(o_ref, best * m_ref[...], _H)


def _pool2_kernel(x_ref, m_ref, o_ref, *, rows, d, G):
    j = pl.program_id(0)

    @pl.when((j == 0) | (j == G + 1))
    def _():
        o_ref[...] = jnp.zeros(o_ref.shape, o_ref.dtype)

    @pl.when((j > 0) & (j < G + 1))
    def _():
        t0 = (j - 1) * rows
        best = jnp.full((rows, _H, x_ref.shape[-1]), _NEG, _F)
        for dy in (0, d):
            for dx in (0, d):
                best = jnp.maximum(
                    best,
                    x_ref[pl.ds(t0 + dy, rows), dx:dx + _H, :].astype(_F))
        best = jnp.where(best < -1e29, 0.0, best)
        _store_padded(o_ref, best * m_ref[...], _H)


def _full(shape):
    n = len(shape)
    return pl.BlockSpec(shape, lambda j: (0,) * n)


def _taps(w):
    return jnp.transpose(w, (2, 3, 1, 0)).reshape(9, w.shape[1], w.shape[0])


def _padded_call(kern, ins, specs, *, rows, out_h, out_w, cout, out_dt=_F):
    G = out_h // rows
    return pl.pallas_call(
        kern,
        grid=(G + 2,),
        in_specs=specs,
        out_specs=pl.BlockSpec((rows, out_w + 2 * _CP, cout),
                               lambda j: (j, 0, 0)),
        out_shape=jax.ShapeDtypeStruct(
            (out_h + 2 * rows, out_w + 2 * _CP, cout), out_dt),
    )(*ins)


def _mrow_spec(rows):
    return pl.BlockSpec((rows, _H, 1), lambda j: (j, 0, 0))


def kernel(batch, pooling_mask, w1, b1, w2, b2, w3, b3, w4, b4, w5, b5,
           w6, b6, w7, b7, w8, b8, w9, b9, w10, b10):
    mask3 = pooling_mask[0].astype(_F)[:, :, None]
    mp16 = jnp.pad(mask3, ((16, 16), (0, 0), (0, 0)))

    x0p = jnp.pad(batch[0], ((0, 0), (1, 1), (1, 1)))
    x1pl = pl.pallas_call(
        _conv1_kernel,
        in_specs=[pl.BlockSpec(memory_space=pltpu.VMEM),
                  pl.BlockSpec(memory_space=pltpu.SMEM),
                  pl.BlockSpec(memory_space=pltpu.SMEM)],
        out_specs=pl.BlockSpec(memory_space=pltpu.VMEM),
        out_shape=jax.ShapeDtypeStruct((64, 224, 224), jnp.bfloat16),
    )(x0p, w1.reshape(64, 27), b1)
    x1p = jnp.pad(jnp.transpose(x1pl, (1, 2, 0)),
                  ((1, 1), (_CP, _CP), (0, 0)))

    x2f = _padded_call(
        functools.partial(_dconv_kernel, rows=16, out_w=224, d=1, in_rp=1,
                          G=14),
        (x1p, _taps(w2), b2.reshape(1, -1)),
        [_full(x1p.shape), _full((9, 64, 64)), _full((1, 64))],
        rows=16, out_h=224, out_w=224, cout=64)

    x2 = _padded_call(
        functools.partial(_maxpool_kernel, rows=16, in_rp=16, G=7),
        (x2f,), [_full(x2f.shape)],
        rows=16, out_h=_H, out_w=_H, cout=64,
        out_dt=jnp.bfloat16)

    x3 = _padded_call(
        functools.partial(_dconv_kernel, rows=16, out_w=_H, d=1, in_rp=16,
                          G=7),
        (x2, _taps(w3), b3.reshape(1, -1)),
        [_full(x2.shape), _full((9, 64, 128)), _full((1, 128))],
        rows=16, out_h=_H, out_w=_H, cout=128,
        out_dt=jnp.bfloat16)

    x4 = _padded_call(
        functools.partial(_dconv_sent_kernel, rows=16, d=1, in_rp=16, G=7),
        (x3, _taps(w4), b4.reshape(1, -1), mp16),
        [_full(x3.shape), _full((9, 128, 128)), _full((1, 128)),
         _mrow_spec(16)],
        rows=16, out_h=_H, out_w=_H, cout=128,
        out_dt=jnp.bfloat16)

    n1 = _padded_call(
        functools.partial(_pool1_kernel, rows=16, d=1, in_rp=16, G=7),
        (x4, mp16), [_full(x4.shape), _mrow_spec(16)],
        rows=16, out_h=_H, out_w=_H, cout=128,
        out_dt=jnp.bfloat16)

    def gconv_pad(x, w, b, mp, rows, d, in_rp, cin, cout, mmdt=_F,
                  sentinel=False, out_dt=_F):
        return _padded_call(
            functools.partial(_gconv_kernel, rows=rows, d=d, in_rp=in_rp,
                              G=_H // rows, mmdt=mmdt, sentinel=sentinel),
            (x, w.astype(mmdt), b.reshape(1, -1), mp),
            [_full(x.shape), _full((9, cin, cout)), _full((1, cout)),
             _mrow_spec(rows)],
            rows=rows, out_h=_H, out_w=_H, cout=cout, out_dt=out_dt)

    bf = jnp.bfloat16
    g5 = gconv_pad(n1, w5, b5, mp16, 16, 2, 16, 128, 256, out_dt=bf)
    g6 = gconv_pad(g5, w6, b6, mp16, 16, 2, 16, 256, 256, out_dt=bf)
    g7 = gconv_pad(g6, w7, b7, mp16, 16, 2, 16, 256, 256, sentinel=True,
                   out_dt=bf)

    g7i = g7[16:16 + _H, _CP:_CP + _H, :]
    g7e = jnp.pad(g7i, ((0, 8), (0, 8), (0, 0)), mode="edge")
    n2 = _padded_call(
        functools.partial(_pool2_kernel, rows=16, d=2, G=7),
        (g7e, mp16), [_full(g7e.shape), _mrow_spec(16)],
        rows=16, out_h=_H, out_w=_H, cout=256,
        out_dt=jnp.bfloat16)

    g8 = gconv_pad(n2, w8, b8, mp16, 16, 4, 16, 256, 512, out_dt=bf)
    g9 = gconv_pad(g8, w9, b9, mp16, 16, 4, 16, 512, 512,
                   mmdt=jnp.bfloat16, out_dt=jnp.bfloat16)

    g10 = pl.pallas_call(
        functools.partial(_gconv_last_kernel, rows=16, d=4, in_rp=16,
                          mmdt=jnp.bfloat16),
        grid=(7,),
        in_specs=[_full(g9.shape), _full((9, 512, 512)), _full((1, 512)),
                  pl.BlockSpec((16, _H, 1), lambda i: (i, 0, 0))],
        out_specs=pl.BlockSpec((16, _H, 512), lambda i: (i, 0, 0)),
        out_shape=jax.ShapeDtypeStruct((_H, _H, 512), _F),
    )(g9, w10.astype(jnp.bfloat16), b10.reshape(1, -1), mask3)

    return jnp.transpose(g10, (2, 0, 1))[None]
